# baseline (speedup 1.0000x reference)
import functools

import jax
import jax.numpy as jnp
from jax import lax
from jax.experimental import pallas as pl
from jax.experimental.pallas import tpu as pltpu
from jax.experimental.pallas import tpu_sc as plsc

_N = 100000
_C = 64
_B = 2
_G = 128
_GP = _G + 4
_NPAD = 102400
_ROWS = _NPAD // 128
_TSIZE = 4_608_000
_PSAFE = ((0 * _GP + 2) * _GP + 2) * _GP + 2
_DMAX = _PSAFE

_NC = 2
_NS = 16


def _prep_body(f_ref, b_ref, z_ref, y_ref, x_ref,
               fa_ref, fm_ref, ps_ref, pg_ref):
  i = pl.program_id(0)
  f = f_ref[...]
  fa_ref[...] = jnp.mean(f, axis=1).reshape(8, 128)
  fm_ref[...] = jnp.max(f, axis=1).reshape(8, 128)
  r = lax.broadcasted_iota(jnp.int32, (8, 128), 0)
  c = lax.broadcasted_iota(jnp.int32, (8, 128), 1)
  vid = (i * 8 + r) * 128 + c
  p = ((b_ref[...] * _GP + z_ref[...] + 2) * _GP
       + y_ref[...] + 2) * _GP + x_ref[...] + 2
  valid = vid < _N
  ps_ref[...] = jnp.where(valid, p, 0)
  pg_ref[...] = jnp.where(valid, p, _PSAFE)


def _prep(feats_pad, b2, z2, y2, x2):
  coord_spec = pl.BlockSpec((8, 128), lambda i: (i, 0))
  return pl.pallas_call(
      _prep_body,
      grid=(100,),
      in_specs=[pl.BlockSpec((1024, 64), lambda i: (i, 0)),
                coord_spec, coord_spec, coord_spec, coord_spec],
      out_specs=[coord_spec, coord_spec, coord_spec, coord_spec],
      out_shape=[
          jax.ShapeDtypeStruct((_ROWS, 128), jnp.float32),
          jax.ShapeDtypeStruct((_ROWS, 128), jnp.float32),
          jax.ShapeDtypeStruct((_ROWS, 128), jnp.int32),
          jax.ShapeDtypeStruct((_ROWS, 128), jnp.int32),
      ],
  )(feats_pad, b2, z2, y2, x2)


_ZCHUNK = 12000


def _scatter_body(ps_hbm, fp_hbm, gp_hbm, zbuf, idx_v, fp_v, sem):
  tid = lax.axis_index("s")
  ne = 2 * _NPAD // _NS
  nwords = 2 * _TSIZE // _NS
  nchunk = nwords // _ZCHUNK

  def zfill(t, carry):
    zbuf[pl.ds(16 * t, 16)] = jnp.zeros((16,), jnp.float32)
    return carry
  lax.fori_loop(0, _ZCHUNK // 16, zfill, 0)

  base = tid * nwords

  def zissue(t, carry):
    pltpu.async_copy(zbuf, gp_hbm.at[pl.ds(base + t * _ZCHUNK, _ZCHUNK)], sem)
    return carry
  lax.fori_loop(0, nchunk, zissue, 0)

  def zdrain(t, carry):
    pltpu.make_async_copy(
        zbuf, gp_hbm.at[pl.ds(base + t * _ZCHUNK, _ZCHUNK)], sem).wait()
    return carry
  lax.fori_loop(0, nchunk, zdrain, 0)

  plsc.subcore_barrier()

  v0 = tid * ne
  pltpu.sync_copy(ps_hbm.at[pl.ds(v0, ne)], idx_v)
  pltpu.sync_copy(fp_hbm.at[pl.ds(v0, ne)], fp_v)
  pltpu.async_copy(fp_v, gp_hbm.at[idx_v], sem).wait()


def _scatter(psi, fpi):
  mesh = plsc.VectorSubcoreMesh(
      core_axis_name="c", subcore_axis_name="s", num_cores=1)
  ne = 2 * _NPAD // _NS
  return pl.kernel(
      _scatter_body,
      out_type=jax.ShapeDtypeStruct((2 * _TSIZE,), jnp.float32),
      mesh=mesh,
      scratch_types=[
          pltpu.VMEM((_ZCHUNK,), jnp.float32),
          pltpu.VMEM((ne,), jnp.int32),
          pltpu.VMEM((ne,), jnp.float32),
          pltpu.SemaphoreType.DMA,
      ],
  )(psi, fpi)


def _gather_body(gp_hbm, pg_hbm, wp_hbm, acc_hbm,
                 pb_v, idx_v, g_v, accp_v, wp_v, sem):
  wid = lax.axis_index("s") * _NC + lax.axis_index("c")
  ne = 2 * _NPAD // (_NC * _NS)
  ng = ne // 16
  v0 = wid * ne
  pltpu.sync_copy(pg_hbm.at[pl.ds(v0, ne)], pb_v)
  pltpu.sync_copy(wp_hbm, wp_v)

  def azero(t, carry):
    accp_v[pl.ds(16 * t, 16)] = jnp.zeros((16,), jnp.float32)
    return carry
  lax.fori_loop(0, ng, azero, 0)

  def kbody(k, carry):
    dz = k // 25 - 2
    dy = (k // 5) % 5 - 2
    dx = k % 5 - 2
    d = 2 * ((dz * _GP + dy) * _GP + dx)

    def tbody(t, c2):
      s = pl.ds(16 * t, 16)
      idx_v[s] = pb_v[s] + d
      return c2
    lax.fori_loop(0, ng, tbody, 0)

    cp = pltpu.async_copy(gp_hbm.at[idx_v], g_v, sem)
    wp = wp_v[k]
    cp.wait()

    def tb(t, c2):
      s = pl.ds(16 * t, 16)
      accp_v[s] = accp_v[s] + wp * g_v[s]
      return c2
    lax.fori_loop(0, ng, tb, 0)
    return carry
  lax.fori_loop(0, 125, kbody, 0)

  pltpu.sync_copy(accp_v, acc_hbm.at[pl.ds(v0, ne)])


def _gather(gflat, pgi, wpt):
  mesh = plsc.VectorSubcoreMesh(core_axis_name="c", subcore_axis_name="s")
  ne = 2 * _NPAD // (_NC * _NS)
  return pl.kernel(
      _gather_body,
      out_type=jax.ShapeDtypeStruct((2 * _NPAD,), jnp.float32),
      mesh=mesh,
      scratch_types=[
          pltpu.VMEM((ne,), jnp.int32),
          pltpu.VMEM((ne,), jnp.int32),
          pltpu.VMEM((ne,), jnp.float32),
          pltpu.VMEM((ne,), jnp.float32),
          pltpu.VMEM((128, 16), jnp.float32),
          pltpu.SemaphoreType.DMA,
      ],
  )(gflat, pgi, wpt)


def _gate_body(f_ref, a_ref, o_ref):
  r = lax.broadcasted_iota(jnp.int32, (256, 128), 0)
  c = lax.broadcasted_iota(jnp.int32, (256, 128), 1)
  sel = (r // 2 == c).astype(jnp.float32)
  acc = lax.dot_general(a_ref[...], sel, (((1,), (0,)), ((), ())),
                        preferred_element_type=jnp.float32)
  g8 = 1.0 / (1.0 + jnp.exp(-acc))
  ones = jnp.ones((1, _C), jnp.float32)
  for s in range(8):
    gcol = lax.dot_general(g8[s:s + 1, :], ones, (((0,), (0,)), ((), ())),
                           preferred_element_type=jnp.float32)
    rs = pl.ds(s * 128, 128)
    o_ref[rs, :] = f_ref[rs, :] * gcol


def _gate(feats_pad, acci):
  return pl.pallas_call(
      _gate_body,
      grid=(100,),
      in_specs=[pl.BlockSpec((1024, 64), lambda i: (i, 0)),
                pl.BlockSpec((8, 256), lambda i: (i, 0))],
      out_specs=pl.BlockSpec((1024, 64), lambda i: (i, 0)),
      out_shape=jax.ShapeDtypeStruct((_NPAD, _C), jnp.float32),
  )(feats_pad, acci)


def kernel(features, indices, W):
  n = features.shape[0]
  pad = _NPAD - n
  feats_pad = jnp.pad(features, ((0, pad), (0, 0)))
  b2 = jnp.pad(indices[:, 0], (0, pad)).reshape(_ROWS, 128)
  z2 = jnp.pad(indices[:, 1], (0, pad)).reshape(_ROWS, 128)
  y2 = jnp.pad(indices[:, 2], (0, pad)).reshape(_ROWS, 128)
  x2 = jnp.pad(indices[:, 3], (0, pad)).reshape(_ROWS, 128)

  wpt = jnp.pad(jnp.tile(W[:, :, 0], (1, 8)), ((0, 3), (0, 0)))

  fa2, fm2, ps2, pg2 = _prep(feats_pad, b2, z2, y2, x2)
  fpi = jnp.stack([fa2.reshape(-1), fm2.reshape(-1)], axis=-1).reshape(-1)
  ps = ps2.reshape(-1)
  psi = jnp.stack([2 * ps, 2 * ps + 1], axis=-1).reshape(-1)
  pg = pg2.reshape(-1)
  pgi = jnp.stack([2 * pg, 2 * pg + 1], axis=-1).reshape(-1)
  gflat = _scatter(psi, fpi)
  acci = _gather(gflat, pgi, wpt)
  out = _gate(feats_pad, acci.reshape(_ROWS, 256))
  return out[:n]


# speedup vs baseline: 1.3900x; 1.3900x over previous
import functools

import jax
import jax.numpy as jnp
from jax import lax
from jax.experimental import pallas as pl
from jax.experimental.pallas import tpu as pltpu
from jax.experimental.pallas import tpu_sc as plsc

_N = 100000
_C = 64
_B = 2
_G = 128
_GP = _G + 4
_NPAD = 102400
_ROWS = _NPAD // 128
_TSIZE = 4_608_000
_PSAFE = ((0 * _GP + 2) * _GP + 2) * _GP + 2

_NC = 2
_NS = 16


def _prep_body(f_ref, b_ref, z_ref, y_ref, x_ref,
               fa_ref, fm_ref, ps_ref, pg_ref):
  i = pl.program_id(0)
  f = f_ref[...]
  fa_ref[...] = jnp.mean(f, axis=1).reshape(8, 128)
  fm_ref[...] = jnp.max(f, axis=1).reshape(8, 128)
  r = lax.broadcasted_iota(jnp.int32, (8, 128), 0)
  c = lax.broadcasted_iota(jnp.int32, (8, 128), 1)
  vid = (i * 8 + r) * 128 + c
  p = ((b_ref[...] * _GP + z_ref[...] + 2) * _GP
       + y_ref[...] + 2) * _GP + x_ref[...] + 2
  valid = vid < _N
  ps_ref[...] = jnp.where(valid, p, 0)
  pg_ref[...] = jnp.where(valid, p, _PSAFE)


def _prep(feats_pad, b2, z2, y2, x2):
  coord_spec = pl.BlockSpec((8, 128), lambda i: (i, 0))
  return pl.pallas_call(
      _prep_body,
      grid=(100,),
      in_specs=[pl.BlockSpec((1024, 64), lambda i: (i, 0)),
                coord_spec, coord_spec, coord_spec, coord_spec],
      out_specs=[coord_spec, coord_spec, coord_spec, coord_spec],
      out_shape=[
          jax.ShapeDtypeStruct((_ROWS, 128), jnp.float32),
          jax.ShapeDtypeStruct((_ROWS, 128), jnp.float32),
          jax.ShapeDtypeStruct((_ROWS, 128), jnp.int32),
          jax.ShapeDtypeStruct((_ROWS, 128), jnp.int32),
      ],
  )(feats_pad, b2, z2, y2, x2)


_ZCHUNK = 12000


def _scatter_body(ps_hbm, fa_hbm, fm_hbm, ga_hbm, gm_hbm,
                  zbuf, idx_v, fa_v, fm_v, sem):
  tid = lax.axis_index("s")
  nv = _NPAD // _NS
  nwords = _TSIZE // _NS
  nchunk = nwords // _ZCHUNK

  def zfill(t, carry):
    zbuf[pl.ds(16 * t, 16)] = jnp.zeros((16,), jnp.float32)
    return carry
  lax.fori_loop(0, _ZCHUNK // 16, zfill, 0)

  base = tid * nwords

  def zissue(t, carry):
    pltpu.async_copy(zbuf, ga_hbm.at[pl.ds(base + t * _ZCHUNK, _ZCHUNK)], sem)
    pltpu.async_copy(zbuf, gm_hbm.at[pl.ds(base + t * _ZCHUNK, _ZCHUNK)], sem)
    return carry
  lax.fori_loop(0, nchunk, zissue, 0)

  def zdrain(t, carry):
    pltpu.make_async_copy(
        zbuf, ga_hbm.at[pl.ds(base + t * _ZCHUNK, _ZCHUNK)], sem).wait()
    pltpu.make_async_copy(
        zbuf, gm_hbm.at[pl.ds(base + t * _ZCHUNK, _ZCHUNK)], sem).wait()
    return carry
  lax.fori_loop(0, nchunk, zdrain, 0)

  plsc.subcore_barrier()

  v0 = tid * nv
  pltpu.sync_copy(ps_hbm.at[pl.ds(v0, nv)], idx_v)
  pltpu.sync_copy(fa_hbm.at[pl.ds(v0, nv)], fa_v)
  pltpu.sync_copy(fm_hbm.at[pl.ds(v0, nv)], fm_v)
  ca = pltpu.async_copy(fa_v, ga_hbm.at[idx_v], sem)
  cm = pltpu.async_copy(fm_v, gm_hbm.at[idx_v], sem)
  ca.wait()
  cm.wait()


def _scatter(ps1, fa1, fm1):
  mesh = plsc.VectorSubcoreMesh(
      core_axis_name="c", subcore_axis_name="s", num_cores=1)
  nv = _NPAD // _NS
  return pl.kernel(
      _scatter_body,
      out_type=[jax.ShapeDtypeStruct((_TSIZE,), jnp.float32),
                jax.ShapeDtypeStruct((_TSIZE,), jnp.float32)],
      mesh=mesh,
      scratch_types=[
          pltpu.VMEM((_ZCHUNK,), jnp.float32),
          pltpu.VMEM((nv,), jnp.int32),
          pltpu.VMEM((nv,), jnp.float32),
          pltpu.VMEM((nv,), jnp.float32),
          pltpu.SemaphoreType.DMA,
      ],
  )(ps1, fa1, fm1)


def _gather_body(ga_hbm, gm_hbm, pg_hbm, w0_hbm, w1_hbm, acc_hbm,
                 pb_v, ia_v, ib_v, gaa_v, gma_v, gab_v, gmb_v,
                 acc_v, w0_v, w1_v, sema, semb):
  wid = lax.axis_index("s") * _NC + lax.axis_index("c")
  nv = _NPAD // (_NC * _NS)
  ng = nv // 16
  v0 = wid * nv
  pltpu.sync_copy(pg_hbm.at[pl.ds(v0, nv)], pb_v)
  pltpu.sync_copy(w0_hbm, w0_v)
  pltpu.sync_copy(w1_hbm, w1_v)

  def azero(t, carry):
    acc_v[pl.ds(16 * t, 16)] = jnp.zeros((16,), jnp.float32)
    return carry
  lax.fori_loop(0, ng, azero, 0)

  def build(k, ivec):
    dz = k // 25 - 2
    dy = (k // 5) % 5 - 2
    dx = k % 5 - 2
    d = (dz * _GP + dy) * _GP + dx

    def tbody(t, c2):
      s = pl.ds(16 * t, 16)
      ivec[s] = pb_v[s] + d
      return c2
    lax.fori_loop(0, ng, tbody, 0)

  def issue(ivec, gav, gmv, sem):
    pltpu.async_copy(ga_hbm.at[ivec], gav, sem)
    pltpu.async_copy(gm_hbm.at[ivec], gmv, sem)

  def drain(ivec, gav, gmv, sem):
    pltpu.make_async_copy(ga_hbm.at[ivec], gav, sem).wait()
    pltpu.make_async_copy(gm_hbm.at[ivec], gmv, sem).wait()

  def accum(k, gav, gmv):
    w0 = w0_v[k]
    w1 = w1_v[k]

    def tb(t, c2):
      s = pl.ds(16 * t, 16)
      acc_v[s] = acc_v[s] + w0 * gav[s] + w1 * gmv[s]
      return c2
    lax.fori_loop(0, ng, tb, 0)

  build(0, ia_v)
  issue(ia_v, gaa_v, gma_v, sema)

  def kbody(i2, carry):
    k0 = 2 * i2
    k1 = k0 + 1

    @pl.when(k1 < 125)
    def _():
      build(k1, ib_v)
      issue(ib_v, gab_v, gmb_v, semb)

    drain(ia_v, gaa_v, gma_v, sema)
    accum(k0, gaa_v, gma_v)

    k2 = k0 + 2

    @pl.when(k2 < 125)
    def _():
      build(k2, ia_v)
      issue(ia_v, gaa_v, gma_v, sema)

    @pl.when(k1 < 125)
    def _():
      drain(ib_v, gab_v, gmb_v, semb)
      accum(k1, gab_v, gmb_v)
    return carry
  lax.fori_loop(0, 63, kbody, 0)

  pltpu.sync_copy(acc_v, acc_hbm.at[pl.ds(v0, nv)])


def _gather(ga, gm, pg1, w0t, w1t):
  mesh = plsc.VectorSubcoreMesh(core_axis_name="c", subcore_axis_name="s")
  nv = _NPAD // (_NC * _NS)
  return pl.kernel(
      _gather_body,
      out_type=jax.ShapeDtypeStruct((_NPAD,), jnp.float32),
      mesh=mesh,
      scratch_types=[
          pltpu.VMEM((nv,), jnp.int32),
          pltpu.VMEM((nv,), jnp.int32),
          pltpu.VMEM((nv,), jnp.int32),
          pltpu.VMEM((nv,), jnp.float32),
          pltpu.VMEM((nv,), jnp.float32),
          pltpu.VMEM((nv,), jnp.float32),
          pltpu.VMEM((nv,), jnp.float32),
          pltpu.VMEM((nv,), jnp.float32),
          pltpu.VMEM((128, 16), jnp.float32),
          pltpu.VMEM((128, 16), jnp.float32),
          pltpu.SemaphoreType.DMA,
          pltpu.SemaphoreType.DMA,
      ],
  )(ga, gm, pg1, w0t, w1t)


def _gate_body(f_ref, a_ref, o_ref):
  g8 = 1.0 / (1.0 + jnp.exp(-a_ref[...]))
  ones = jnp.ones((1, _C), jnp.float32)
  for s in range(8):
    gcol = lax.dot_general(g8[s:s + 1, :], ones, (((0,), (0,)), ((), ())),
                           preferred_element_type=jnp.float32)
    rs = pl.ds(s * 128, 128)
    o_ref[rs, :] = f_ref[rs, :] * gcol


def _gate(feats_pad, acc2):
  return pl.pallas_call(
      _gate_body,
      grid=(100,),
      in_specs=[pl.BlockSpec((1024, 64), lambda i: (i, 0)),
                pl.BlockSpec((8, 128), lambda i: (i, 0))],
      out_specs=pl.BlockSpec((1024, 64), lambda i: (i, 0)),
      out_shape=jax.ShapeDtypeStruct((_NPAD, _C), jnp.float32),
  )(feats_pad, acc2)


def kernel(features, indices, W):
  n = features.shape[0]
  pad = _NPAD - n
  feats_pad = jnp.pad(features, ((0, pad), (0, 0)))
  b2 = jnp.pad(indices[:, 0], (0, pad)).reshape(_ROWS, 128)
  z2 = jnp.pad(indices[:, 1], (0, pad)).reshape(_ROWS, 128)
  y2 = jnp.pad(indices[:, 2], (0, pad)).reshape(_ROWS, 128)
  x2 = jnp.pad(indices[:, 3], (0, pad)).reshape(_ROWS, 128)

  w0t = jnp.pad(jnp.broadcast_to(W[:, 0, :], (125, 16)), ((0, 3), (0, 0)))
  w1t = jnp.pad(jnp.broadcast_to(W[:, 1, :], (125, 16)), ((0, 3), (0, 0)))

  fa2, fm2, ps2, pg2 = _prep(feats_pad, b2, z2, y2, x2)
  ga, gm = _scatter(ps2.reshape(-1), fa2.reshape(-1), fm2.reshape(-1))
  acc1 = _gather(ga, gm, pg2.reshape(-1), w0t, w1t)
  out = _gate(feats_pad, acc1.reshape(_ROWS, 128))
  return out[:n]


# speedup vs baseline: 1.4590x; 1.0496x over previous
import functools

import jax
import jax.numpy as jnp
from jax import lax
from jax.experimental import pallas as pl
from jax.experimental.pallas import tpu as pltpu
from jax.experimental.pallas import tpu_sc as plsc

_N = 100000
_C = 64
_B = 2
_G = 128
_GP = _G + 4
_NPAD = 102400
_ROWS = _NPAD // 128
_TSIZE = 4_608_000
_PSAFE = ((0 * _GP + 2) * _GP + 2) * _GP + 2

_NC = 2
_NS = 16


def _prep_body(f_ref, b_ref, z_ref, y_ref, x_ref,
               fa_ref, fm_ref, ps_ref, pg_ref):
  i = pl.program_id(0)
  f = f_ref[...]
  fa_ref[...] = jnp.mean(f, axis=1).reshape(8, 128)
  fm_ref[...] = jnp.max(f, axis=1).reshape(8, 128)
  r = lax.broadcasted_iota(jnp.int32, (8, 128), 0)
  c = lax.broadcasted_iota(jnp.int32, (8, 128), 1)
  vid = (i * 8 + r) * 128 + c
  p = ((b_ref[...] * _GP + z_ref[...] + 2) * _GP
       + y_ref[...] + 2) * _GP + x_ref[...] + 2
  valid = vid < _N
  ps_ref[...] = jnp.where(valid, p, 0)
  pg_ref[...] = jnp.where(valid, p, _PSAFE)


def _prep(feats_pad, b2, z2, y2, x2):
  coord_spec = pl.BlockSpec((8, 128), lambda i: (i, 0))
  return pl.pallas_call(
      _prep_body,
      grid=(100,),
      in_specs=[pl.BlockSpec((1024, 64), lambda i: (i, 0)),
                coord_spec, coord_spec, coord_spec, coord_spec],
      out_specs=[coord_spec, coord_spec, coord_spec, coord_spec],
      out_shape=[
          jax.ShapeDtypeStruct((_ROWS, 128), jnp.float32),
          jax.ShapeDtypeStruct((_ROWS, 128), jnp.float32),
          jax.ShapeDtypeStruct((_ROWS, 128), jnp.int32),
          jax.ShapeDtypeStruct((_ROWS, 128), jnp.int32),
      ],
  )(feats_pad, b2, z2, y2, x2)


_ZCHUNK = 12000


def _scatter_body(ps_hbm, fa_hbm, fm_hbm, ga_hbm, gm_hbm,
                  zbuf, idx_v, fa_v, fm_v, sem):
  tid = lax.axis_index("s")
  nv = _NPAD // _NS
  nwords = _TSIZE // _NS
  nchunk = nwords // _ZCHUNK

  def zfill(t, carry):
    zbuf[pl.ds(16 * t, 16)] = jnp.zeros((16,), jnp.float32)
    return carry
  lax.fori_loop(0, _ZCHUNK // 16, zfill, 0)

  base = tid * nwords

  def zissue(t, carry):
    pltpu.async_copy(zbuf, ga_hbm.at[pl.ds(base + t * _ZCHUNK, _ZCHUNK)], sem)
    pltpu.async_copy(zbuf, gm_hbm.at[pl.ds(base + t * _ZCHUNK, _ZCHUNK)], sem)
    return carry
  lax.fori_loop(0, nchunk, zissue, 0)

  def zdrain(t, carry):
    pltpu.make_async_copy(
        zbuf, ga_hbm.at[pl.ds(base + t * _ZCHUNK, _ZCHUNK)], sem).wait()
    pltpu.make_async_copy(
        zbuf, gm_hbm.at[pl.ds(base + t * _ZCHUNK, _ZCHUNK)], sem).wait()
    return carry
  lax.fori_loop(0, nchunk, zdrain, 0)

  plsc.subcore_barrier()

  v0 = tid * nv
  pltpu.sync_copy(ps_hbm.at[pl.ds(v0, nv)], idx_v)
  pltpu.sync_copy(fa_hbm.at[pl.ds(v0, nv)], fa_v)
  pltpu.sync_copy(fm_hbm.at[pl.ds(v0, nv)], fm_v)
  ca = pltpu.async_copy(fa_v, ga_hbm.at[idx_v], sem)
  cm = pltpu.async_copy(fm_v, gm_hbm.at[idx_v], sem)
  ca.wait()
  cm.wait()


def _scatter(ps1, fa1, fm1):
  mesh = plsc.VectorSubcoreMesh(
      core_axis_name="c", subcore_axis_name="s", num_cores=1)
  nv = _NPAD // _NS
  return pl.kernel(
      _scatter_body,
      out_type=[jax.ShapeDtypeStruct((_TSIZE,), jnp.float32),
                jax.ShapeDtypeStruct((_TSIZE,), jnp.float32)],
      mesh=mesh,
      scratch_types=[
          pltpu.VMEM((_ZCHUNK,), jnp.float32),
          pltpu.VMEM((nv,), jnp.int32),
          pltpu.VMEM((nv,), jnp.float32),
          pltpu.VMEM((nv,), jnp.float32),
          pltpu.SemaphoreType.DMA,
      ],
  )(ps1, fa1, fm1)


def _gather_body(ga_hbm, gm_hbm, pg_hbm, w0_hbm, w1_hbm, acc_hbm,
                 pb_v, ia_v, ib_v, gaa_v, gma_v, gab_v, gmb_v,
                 acc_v, w0_v, w1_v, sema, semb):
  wid = lax.axis_index("s") * _NC + lax.axis_index("c")
  nv = _NPAD // (_NC * _NS)
  ng = nv // 16
  v0 = wid * nv
  pltpu.sync_copy(pg_hbm.at[pl.ds(v0, nv)], pb_v)
  pltpu.sync_copy(w0_hbm, w0_v)
  pltpu.sync_copy(w1_hbm, w1_v)

  def azero(t, carry):
    acc_v[pl.ds(16 * t, 16)] = jnp.zeros((16,), jnp.float32)
    return carry
  lax.fori_loop(0, ng, azero, 0)

  def build(k, ivec):
    dz = k // 25 - 2
    dy = (k // 5) % 5 - 2
    dx = k % 5 - 2
    d = (dz * _GP + dy) * _GP + dx

    def tbody(t, c2):
      s = pl.ds(16 * t, 16)
      ivec[s] = pb_v[s] + d
      return c2
    lax.fori_loop(0, ng, tbody, 0)

  nsub = 4
  sub = nv // nsub

  def issue(ivec, gav, gmv, sem):
    for u in range(nsub):
      s = pl.ds(u * sub, sub)
      pltpu.async_copy(ga_hbm.at[ivec.at[s]], gav.at[s], sem)
      pltpu.async_copy(gm_hbm.at[ivec.at[s]], gmv.at[s], sem)

  def drain(ivec, gav, gmv, sem):
    for u in range(nsub):
      s = pl.ds(u * sub, sub)
      pltpu.make_async_copy(ga_hbm.at[ivec.at[s]], gav.at[s], sem).wait()
      pltpu.make_async_copy(gm_hbm.at[ivec.at[s]], gmv.at[s], sem).wait()

  def accum(k, gav, gmv):
    w0 = w0_v[k]
    w1 = w1_v[k]

    def tb(t, c2):
      s = pl.ds(16 * t, 16)
      acc_v[s] = acc_v[s] + w0 * gav[s] + w1 * gmv[s]
      return c2
    lax.fori_loop(0, ng, tb, 0)

  build(0, ia_v)
  issue(ia_v, gaa_v, gma_v, sema)

  def kbody(i2, carry):
    k0 = 2 * i2
    k1 = k0 + 1

    @pl.when(k1 < 125)
    def _():
      build(k1, ib_v)
      issue(ib_v, gab_v, gmb_v, semb)

    drain(ia_v, gaa_v, gma_v, sema)
    accum(k0, gaa_v, gma_v)

    k2 = k0 + 2

    @pl.when(k2 < 125)
    def _():
      build(k2, ia_v)
      issue(ia_v, gaa_v, gma_v, sema)

    @pl.when(k1 < 125)
    def _():
      drain(ib_v, gab_v, gmb_v, semb)
      accum(k1, gab_v, gmb_v)
    return carry
  lax.fori_loop(0, 63, kbody, 0)

  pltpu.sync_copy(acc_v, acc_hbm.at[pl.ds(v0, nv)])


def _gather(ga, gm, pg1, w0t, w1t):
  mesh = plsc.VectorSubcoreMesh(core_axis_name="c", subcore_axis_name="s")
  nv = _NPAD // (_NC * _NS)
  return pl.kernel(
      _gather_body,
      out_type=jax.ShapeDtypeStruct((_NPAD,), jnp.float32),
      mesh=mesh,
      scratch_types=[
          pltpu.VMEM((nv,), jnp.int32),
          pltpu.VMEM((nv,), jnp.int32),
          pltpu.VMEM((nv,), jnp.int32),
          pltpu.VMEM((nv,), jnp.float32),
          pltpu.VMEM((nv,), jnp.float32),
          pltpu.VMEM((nv,), jnp.float32),
          pltpu.VMEM((nv,), jnp.float32),
          pltpu.VMEM((nv,), jnp.float32),
          pltpu.VMEM((128, 16), jnp.float32),
          pltpu.VMEM((128, 16), jnp.float32),
          pltpu.SemaphoreType.DMA,
          pltpu.SemaphoreType.DMA,
      ],
  )(ga, gm, pg1, w0t, w1t)


def _gate_body(f_ref, a_ref, o_ref):
  g8 = 1.0 / (1.0 + jnp.exp(-a_ref[...]))
  ones = jnp.ones((1, _C), jnp.float32)
  for s in range(8):
    gcol = lax.dot_general(g8[s:s + 1, :], ones, (((0,), (0,)), ((), ())),
                           preferred_element_type=jnp.float32)
    rs = pl.ds(s * 128, 128)
    o_ref[rs, :] = f_ref[rs, :] * gcol


def _gate(feats_pad, acc2):
  return pl.pallas_call(
      _gate_body,
      grid=(100,),
      in_specs=[pl.BlockSpec((1024, 64), lambda i: (i, 0)),
                pl.BlockSpec((8, 128), lambda i: (i, 0))],
      out_specs=pl.BlockSpec((1024, 64), lambda i: (i, 0)),
      out_shape=jax.ShapeDtypeStruct((_NPAD, _C), jnp.float32),
  )(feats_pad, acc2)


def kernel(features, indices, W):
  n = features.shape[0]
  pad = _NPAD - n
  feats_pad = jnp.pad(features, ((0, pad), (0, 0)))
  b2 = jnp.pad(indices[:, 0], (0, pad)).reshape(_ROWS, 128)
  z2 = jnp.pad(indices[:, 1], (0, pad)).reshape(_ROWS, 128)
  y2 = jnp.pad(indices[:, 2], (0, pad)).reshape(_ROWS, 128)
  x2 = jnp.pad(indices[:, 3], (0, pad)).reshape(_ROWS, 128)

  w0t = jnp.pad(jnp.broadcast_to(W[:, 0, :], (125, 16)), ((0, 3), (0, 0)))
  w1t = jnp.pad(jnp.broadcast_to(W[:, 1, :], (125, 16)), ((0, 3), (0, 0)))

  fa2, fm2, ps2, pg2 = _prep(feats_pad, b2, z2, y2, x2)
  ga, gm = _scatter(ps2.reshape(-1), fa2.reshape(-1), fm2.reshape(-1))
  acc1 = _gather(ga, gm, pg2.reshape(-1), w0t, w1t)
  out = _gate(feats_pad, acc1.reshape(_ROWS, 128))
  return out[:n]


# speedup vs baseline: 1.5749x; 1.0795x over previous
import functools

import jax
import jax.numpy as jnp
from jax import lax
from jax.experimental import pallas as pl
from jax.experimental.pallas import tpu as pltpu
from jax.experimental.pallas import tpu_sc as plsc

_N = 100000
_C = 64
_B = 2
_G = 128
_GP = _G + 4
_NPAD = 102400
_ROWS = _NPAD // 128
_TSIZE = 4_608_000
_PSAFE = ((0 * _GP + 2) * _GP + 2) * _GP + 2

_NC = 2
_NS = 16


def _prep_body(f_ref, b_ref, z_ref, y_ref, x_ref,
               fa_ref, fm_ref, ps_ref, pg_ref):
  i = pl.program_id(0)
  f = f_ref[...]
  fa_ref[...] = jnp.mean(f, axis=1).reshape(8, 128)
  fm_ref[...] = jnp.max(f, axis=1).reshape(8, 128)
  r = lax.broadcasted_iota(jnp.int32, (8, 128), 0)
  c = lax.broadcasted_iota(jnp.int32, (8, 128), 1)
  vid = (i * 8 + r) * 128 + c
  p = ((b_ref[...] * _GP + z_ref[...] + 2) * _GP
       + y_ref[...] + 2) * _GP + x_ref[...] + 2
  valid = vid < _N
  ps_ref[...] = jnp.where(valid, p, 0)
  pg_ref[...] = jnp.where(valid, p, _PSAFE)


def _prep(feats_pad, b2, z2, y2, x2):
  coord_spec = pl.BlockSpec((8, 128), lambda i: (i, 0))
  return pl.pallas_call(
      _prep_body,
      grid=(100,),
      in_specs=[pl.BlockSpec((1024, 64), lambda i: (i, 0)),
                coord_spec, coord_spec, coord_spec, coord_spec],
      out_specs=[coord_spec, coord_spec, coord_spec, coord_spec],
      out_shape=[
          jax.ShapeDtypeStruct((_ROWS, 128), jnp.float32),
          jax.ShapeDtypeStruct((_ROWS, 128), jnp.float32),
          jax.ShapeDtypeStruct((_ROWS, 128), jnp.int32),
          jax.ShapeDtypeStruct((_ROWS, 128), jnp.int32),
      ],
  )(feats_pad, b2, z2, y2, x2)


_ZCHUNK = 12000


def _scatter_body(ps_hbm, fa_hbm, fm_hbm, ga_hbm, gm_hbm,
                  zbuf, idx_v, fa_v, fm_v, sem):
  tid = lax.axis_index("s")
  nv = _NPAD // _NS
  nwords = _TSIZE // _NS
  nchunk = nwords // _ZCHUNK

  def zfill(t, carry):
    zbuf[pl.ds(16 * t, 16)] = jnp.zeros((16,), jnp.float32)
    return carry
  lax.fori_loop(0, _ZCHUNK // 16, zfill, 0)

  base = tid * nwords

  def zissue(t, carry):
    pltpu.async_copy(zbuf, ga_hbm.at[pl.ds(base + t * _ZCHUNK, _ZCHUNK)], sem)
    pltpu.async_copy(zbuf, gm_hbm.at[pl.ds(base + t * _ZCHUNK, _ZCHUNK)], sem)
    return carry
  lax.fori_loop(0, nchunk, zissue, 0)

  def zdrain(t, carry):
    pltpu.make_async_copy(
        zbuf, ga_hbm.at[pl.ds(base + t * _ZCHUNK, _ZCHUNK)], sem).wait()
    pltpu.make_async_copy(
        zbuf, gm_hbm.at[pl.ds(base + t * _ZCHUNK, _ZCHUNK)], sem).wait()
    return carry
  lax.fori_loop(0, nchunk, zdrain, 0)

  plsc.subcore_barrier()

  v0 = tid * nv
  pltpu.sync_copy(ps_hbm.at[pl.ds(v0, nv)], idx_v)
  pltpu.sync_copy(fa_hbm.at[pl.ds(v0, nv)], fa_v)
  pltpu.sync_copy(fm_hbm.at[pl.ds(v0, nv)], fm_v)
  ca = pltpu.async_copy(fa_v, ga_hbm.at[idx_v], sem)
  cm = pltpu.async_copy(fm_v, gm_hbm.at[idx_v], sem)
  ca.wait()
  cm.wait()


def _scatter(ps1, fa1, fm1):
  mesh = plsc.VectorSubcoreMesh(
      core_axis_name="c", subcore_axis_name="s", num_cores=1)
  nv = _NPAD // _NS
  return pl.kernel(
      _scatter_body,
      out_type=[jax.ShapeDtypeStruct((_TSIZE,), jnp.float32),
                jax.ShapeDtypeStruct((_TSIZE,), jnp.float32)],
      mesh=mesh,
      scratch_types=[
          pltpu.VMEM((_ZCHUNK,), jnp.float32),
          pltpu.VMEM((nv,), jnp.int32),
          pltpu.VMEM((nv,), jnp.float32),
          pltpu.VMEM((nv,), jnp.float32),
          pltpu.SemaphoreType.DMA,
      ],
  )(ps1, fa1, fm1)


def _gather_body(ga_hbm, gm_hbm, pg_hbm, w0_hbm, w1_hbm, acc_hbm,
                 pb_v, ia_v, ib_v, gaa_v, gma_v, gab_v, gmb_v,
                 acc_v, w0_v, w1_v, sema, semb):
  wid = lax.axis_index("s") * _NC + lax.axis_index("c")
  nv = _NPAD // (_NC * _NS)
  ng = nv // 16
  v0 = wid * nv
  pltpu.sync_copy(pg_hbm.at[pl.ds(v0, nv)], pb_v)
  pltpu.sync_copy(w0_hbm, w0_v)
  pltpu.sync_copy(w1_hbm, w1_v)

  def azero(t, carry):
    acc_v[pl.ds(16 * t, 16)] = jnp.zeros((16,), jnp.float32)
    return carry
  lax.fori_loop(0, ng, azero, 0)

  def build(k, ivec):
    dz = k // 25 - 2
    dy = (k // 5) % 5 - 2
    dx = k % 5 - 2
    d = (dz * _GP + dy) * _GP + dx

    def tbody(t, c2):
      s = pl.ds(16 * t, 16)
      ivec[s] = pb_v[s] + d
      return c2
    lax.fori_loop(0, ng, tbody, 0)

  nsub = 8
  sub = nv // nsub

  def issue(ivec, gav, gmv, sem):
    for u in range(nsub):
      s = pl.ds(u * sub, sub)
      pltpu.async_copy(ga_hbm.at[ivec.at[s]], gav.at[s], sem)
      pltpu.async_copy(gm_hbm.at[ivec.at[s]], gmv.at[s], sem)

  def drain(ivec, gav, gmv, sem):
    for u in range(nsub):
      s = pl.ds(u * sub, sub)
      pltpu.make_async_copy(ga_hbm.at[ivec.at[s]], gav.at[s], sem).wait()
      pltpu.make_async_copy(gm_hbm.at[ivec.at[s]], gmv.at[s], sem).wait()

  def accum(k, gav, gmv):
    w0 = w0_v[k]
    w1 = w1_v[k]

    def tb(t, c2):
      s = pl.ds(16 * t, 16)
      acc_v[s] = acc_v[s] + w0 * gav[s] + w1 * gmv[s]
      return c2
    lax.fori_loop(0, ng, tb, 0)

  build(0, ia_v)
  issue(ia_v, gaa_v, gma_v, sema)

  def kbody(i2, carry):
    k0 = 2 * i2
    k1 = k0 + 1

    @pl.when(k1 < 125)
    def _():
      build(k1, ib_v)
      issue(ib_v, gab_v, gmb_v, semb)

    drain(ia_v, gaa_v, gma_v, sema)
    accum(k0, gaa_v, gma_v)

    k2 = k0 + 2

    @pl.when(k2 < 125)
    def _():
      build(k2, ia_v)
      issue(ia_v, gaa_v, gma_v, sema)

    @pl.when(k1 < 125)
    def _():
      drain(ib_v, gab_v, gmb_v, semb)
      accum(k1, gab_v, gmb_v)
    return carry
  lax.fori_loop(0, 63, kbody, 0)

  pltpu.sync_copy(acc_v, acc_hbm.at[pl.ds(v0, nv)])


def _gather(ga, gm, pg1, w0t, w1t):
  mesh = plsc.VectorSubcoreMesh(core_axis_name="c", subcore_axis_name="s")
  nv = _NPAD // (_NC * _NS)
  return pl.kernel(
      _gather_body,
      out_type=jax.ShapeDtypeStruct((_NPAD,), jnp.float32),
      mesh=mesh,
      scratch_types=[
          pltpu.VMEM((nv,), jnp.int32),
          pltpu.VMEM((nv,), jnp.int32),
          pltpu.VMEM((nv,), jnp.int32),
          pltpu.VMEM((nv,), jnp.float32),
          pltpu.VMEM((nv,), jnp.float32),
          pltpu.VMEM((nv,), jnp.float32),
          pltpu.VMEM((nv,), jnp.float32),
          pltpu.VMEM((nv,), jnp.float32),
          pltpu.VMEM((128, 16), jnp.float32),
          pltpu.VMEM((128, 16), jnp.float32),
          pltpu.SemaphoreType.DMA,
          pltpu.SemaphoreType.DMA,
      ],
  )(ga, gm, pg1, w0t, w1t)


def _gate_body(f_ref, a_ref, o_ref):
  g8 = 1.0 / (1.0 + jnp.exp(-a_ref[...]))
  ones = jnp.ones((1, _C), jnp.float32)
  for s in range(8):
    gcol = lax.dot_general(g8[s:s + 1, :], ones, (((0,), (0,)), ((), ())),
                           preferred_element_type=jnp.float32)
    rs = pl.ds(s * 128, 128)
    o_ref[rs, :] = f_ref[rs, :] * gcol


def _gate(feats_pad, acc2):
  return pl.pallas_call(
      _gate_body,
      grid=(100,),
      in_specs=[pl.BlockSpec((1024, 64), lambda i: (i, 0)),
                pl.BlockSpec((8, 128), lambda i: (i, 0))],
      out_specs=pl.BlockSpec((1024, 64), lambda i: (i, 0)),
      out_shape=jax.ShapeDtypeStruct((_NPAD, _C), jnp.float32),
  )(feats_pad, acc2)


def kernel(features, indices, W):
  n = features.shape[0]
  pad = _NPAD - n
  feats_pad = jnp.pad(features, ((0, pad), (0, 0)))
  b2 = jnp.pad(indices[:, 0], (0, pad)).reshape(_ROWS, 128)
  z2 = jnp.pad(indices[:, 1], (0, pad)).reshape(_ROWS, 128)
  y2 = jnp.pad(indices[:, 2], (0, pad)).reshape(_ROWS, 128)
  x2 = jnp.pad(indices[:, 3], (0, pad)).reshape(_ROWS, 128)

  w0t = jnp.pad(jnp.broadcast_to(W[:, 0, :], (125, 16)), ((0, 3), (0, 0)))
  w1t = jnp.pad(jnp.broadcast_to(W[:, 1, :], (125, 16)), ((0, 3), (0, 0)))

  fa2, fm2, ps2, pg2 = _prep(feats_pad, b2, z2, y2, x2)
  ga, gm = _scatter(ps2.reshape(-1), fa2.reshape(-1), fm2.reshape(-1))
  acc1 = _gather(ga, gm, pg2.reshape(-1), w0t, w1t)
  out = _gate(feats_pad, acc1.reshape(_ROWS, 128))
  return out[:n]


# speedup vs baseline: 1.6329x; 1.0368x over previous
import functools

import jax
import jax.numpy as jnp
from jax import lax
from jax.experimental import pallas as pl
from jax.experimental.pallas import tpu as pltpu
from jax.experimental.pallas import tpu_sc as plsc

_N = 100000
_C = 64
_B = 2
_G = 128
_GP = _G + 4
_NPAD = 102400
_ROWS = _NPAD // 128
_TSIZE = 4_608_000
_PSAFE = ((0 * _GP + 2) * _GP + 2) * _GP + 2

_NC = 2
_NS = 16


def _prep_body(f_ref, b_ref, z_ref, y_ref, x_ref,
               fa_ref, fm_ref, ps_ref, pg_ref):
  i = pl.program_id(0)
  f = f_ref[...]
  fa_ref[...] = jnp.mean(f, axis=1).reshape(8, 128)
  fm_ref[...] = jnp.max(f, axis=1).reshape(8, 128)
  r = lax.broadcasted_iota(jnp.int32, (8, 128), 0)
  c = lax.broadcasted_iota(jnp.int32, (8, 128), 1)
  vid = (i * 8 + r) * 128 + c
  p = ((b_ref[...] * _GP + z_ref[...] + 2) * _GP
       + y_ref[...] + 2) * _GP + x_ref[...] + 2
  valid = vid < _N
  ps_ref[...] = jnp.where(valid, p, 0)
  pg_ref[...] = jnp.where(valid, p, _PSAFE)


def _prep(feats_pad, b2, z2, y2, x2):
  coord_spec = pl.BlockSpec((8, 128), lambda i: (i, 0))
  return pl.pallas_call(
      _prep_body,
      grid=(100,),
      in_specs=[pl.BlockSpec((1024, 64), lambda i: (i, 0)),
                coord_spec, coord_spec, coord_spec, coord_spec],
      out_specs=[coord_spec, coord_spec, coord_spec, coord_spec],
      out_shape=[
          jax.ShapeDtypeStruct((_ROWS, 128), jnp.float32),
          jax.ShapeDtypeStruct((_ROWS, 128), jnp.float32),
          jax.ShapeDtypeStruct((_ROWS, 128), jnp.int32),
          jax.ShapeDtypeStruct((_ROWS, 128), jnp.int32),
      ],
  )(feats_pad, b2, z2, y2, x2)


_ZCHUNK = 12000


def _scatter_body(ps_hbm, fa_hbm, fm_hbm, ga_hbm, gm_hbm,
                  zbuf, idx_v, fa_v, fm_v, sem):
  tid = lax.axis_index("s")
  nv = _NPAD // _NS
  nwords = _TSIZE // _NS
  nchunk = nwords // _ZCHUNK

  def zfill(t, carry):
    zbuf[pl.ds(16 * t, 16)] = jnp.zeros((16,), jnp.float32)
    return carry
  lax.fori_loop(0, _ZCHUNK // 16, zfill, 0)

  base = tid * nwords

  def zissue(t, carry):
    pltpu.async_copy(zbuf, ga_hbm.at[pl.ds(base + t * _ZCHUNK, _ZCHUNK)], sem)
    pltpu.async_copy(zbuf, gm_hbm.at[pl.ds(base + t * _ZCHUNK, _ZCHUNK)], sem)
    return carry
  lax.fori_loop(0, nchunk, zissue, 0)

  def zdrain(t, carry):
    pltpu.make_async_copy(
        zbuf, ga_hbm.at[pl.ds(base + t * _ZCHUNK, _ZCHUNK)], sem).wait()
    pltpu.make_async_copy(
        zbuf, gm_hbm.at[pl.ds(base + t * _ZCHUNK, _ZCHUNK)], sem).wait()
    return carry
  lax.fori_loop(0, nchunk, zdrain, 0)

  plsc.subcore_barrier()

  v0 = tid * nv
  pltpu.sync_copy(ps_hbm.at[pl.ds(v0, nv)], idx_v)
  pltpu.sync_copy(fa_hbm.at[pl.ds(v0, nv)], fa_v)
  pltpu.sync_copy(fm_hbm.at[pl.ds(v0, nv)], fm_v)
  ca = pltpu.async_copy(fa_v, ga_hbm.at[idx_v], sem)
  cm = pltpu.async_copy(fm_v, gm_hbm.at[idx_v], sem)
  ca.wait()
  cm.wait()


def _scatter(ps1, fa1, fm1):
  mesh = plsc.VectorSubcoreMesh(
      core_axis_name="c", subcore_axis_name="s", num_cores=1)
  nv = _NPAD // _NS
  return pl.kernel(
      _scatter_body,
      out_type=[jax.ShapeDtypeStruct((_TSIZE,), jnp.float32),
                jax.ShapeDtypeStruct((_TSIZE,), jnp.float32)],
      mesh=mesh,
      scratch_types=[
          pltpu.VMEM((_ZCHUNK,), jnp.float32),
          pltpu.VMEM((nv,), jnp.int32),
          pltpu.VMEM((nv,), jnp.float32),
          pltpu.VMEM((nv,), jnp.float32),
          pltpu.SemaphoreType.DMA,
      ],
  )(ps1, fa1, fm1)


def _gather_body(ga_hbm, gm_hbm, pg_hbm, w0_hbm, w1_hbm, acc_hbm,
                 pb_v, ia_v, ib_v, gaa_v, gma_v, gab_v, gmb_v,
                 acc_v, w0_v, w1_v, sema, semb):
  wid = lax.axis_index("s") * _NC + lax.axis_index("c")
  nv = _NPAD // (_NC * _NS)
  ng = nv // 16
  v0 = wid * nv
  pltpu.sync_copy(pg_hbm.at[pl.ds(v0, nv)], pb_v)
  pltpu.sync_copy(w0_hbm, w0_v)
  pltpu.sync_copy(w1_hbm, w1_v)

  def azero(t, carry):
    acc_v[pl.ds(16 * t, 16)] = jnp.zeros((16,), jnp.float32)
    return carry
  lax.fori_loop(0, ng, azero, 0)

  def build(k, ivec):
    dz = k // 25 - 2
    dy = (k // 5) % 5 - 2
    dx = k % 5 - 2
    d = (dz * _GP + dy) * _GP + dx

    def tbody(t, c2):
      s = pl.ds(16 * t, 16)
      ivec[s] = pb_v[s] + d
      return c2
    lax.fori_loop(0, ng, tbody, 0)

  nsub = 16
  sub = nv // nsub

  def issue(ivec, gav, gmv, sem):
    for u in range(nsub):
      s = pl.ds(u * sub, sub)
      pltpu.async_copy(ga_hbm.at[ivec.at[s]], gav.at[s], sem)
      pltpu.async_copy(gm_hbm.at[ivec.at[s]], gmv.at[s], sem)

  def drain(ivec, gav, gmv, sem):
    for u in range(nsub):
      s = pl.ds(u * sub, sub)
      pltpu.make_async_copy(ga_hbm.at[ivec.at[s]], gav.at[s], sem).wait()
      pltpu.make_async_copy(gm_hbm.at[ivec.at[s]], gmv.at[s], sem).wait()

  def accum(k, gav, gmv):
    w0 = w0_v[k]
    w1 = w1_v[k]

    def tb(t, c2):
      s = pl.ds(16 * t, 16)
      acc_v[s] = acc_v[s] + w0 * gav[s] + w1 * gmv[s]
      return c2
    lax.fori_loop(0, ng, tb, 0)

  build(0, ia_v)
  issue(ia_v, gaa_v, gma_v, sema)

  def kbody(i2, carry):
    k0 = 2 * i2
    k1 = k0 + 1

    @pl.when(k1 < 125)
    def _():
      build(k1, ib_v)
      issue(ib_v, gab_v, gmb_v, semb)

    drain(ia_v, gaa_v, gma_v, sema)
    accum(k0, gaa_v, gma_v)

    k2 = k0 + 2

    @pl.when(k2 < 125)
    def _():
      build(k2, ia_v)
      issue(ia_v, gaa_v, gma_v, sema)

    @pl.when(k1 < 125)
    def _():
      drain(ib_v, gab_v, gmb_v, semb)
      accum(k1, gab_v, gmb_v)
    return carry
  lax.fori_loop(0, 63, kbody, 0)

  pltpu.sync_copy(acc_v, acc_hbm.at[pl.ds(v0, nv)])


def _gather(ga, gm, pg1, w0t, w1t):
  mesh = plsc.VectorSubcoreMesh(core_axis_name="c", subcore_axis_name="s")
  nv = _NPAD // (_NC * _NS)
  return pl.kernel(
      _gather_body,
      out_type=jax.ShapeDtypeStruct((_NPAD,), jnp.float32),
      mesh=mesh,
      scratch_types=[
          pltpu.VMEM((nv,), jnp.int32),
          pltpu.VMEM((nv,), jnp.int32),
          pltpu.VMEM((nv,), jnp.int32),
          pltpu.VMEM((nv,), jnp.float32),
          pltpu.VMEM((nv,), jnp.float32),
          pltpu.VMEM((nv,), jnp.float32),
          pltpu.VMEM((nv,), jnp.float32),
          pltpu.VMEM((nv,), jnp.float32),
          pltpu.VMEM((128, 16), jnp.float32),
          pltpu.VMEM((128, 16), jnp.float32),
          pltpu.SemaphoreType.DMA,
          pltpu.SemaphoreType.DMA,
      ],
  )(ga, gm, pg1, w0t, w1t)


def _gate_body(f_ref, a_ref, o_ref):
  g8 = 1.0 / (1.0 + jnp.exp(-a_ref[...]))
  ones = jnp.ones((1, _C), jnp.float32)
  for s in range(8):
    gcol = lax.dot_general(g8[s:s + 1, :], ones, (((0,), (0,)), ((), ())),
                           preferred_element_type=jnp.float32)
    rs = pl.ds(s * 128, 128)
    o_ref[rs, :] = f_ref[rs, :] * gcol


def _gate(feats_pad, acc2):
  return pl.pallas_call(
      _gate_body,
      grid=(100,),
      in_specs=[pl.BlockSpec((1024, 64), lambda i: (i, 0)),
                pl.BlockSpec((8, 128), lambda i: (i, 0))],
      out_specs=pl.BlockSpec((1024, 64), lambda i: (i, 0)),
      out_shape=jax.ShapeDtypeStruct((_NPAD, _C), jnp.float32),
  )(feats_pad, acc2)


def kernel(features, indices, W):
  n = features.shape[0]
  pad = _NPAD - n
  feats_pad = jnp.pad(features, ((0, pad), (0, 0)))
  b2 = jnp.pad(indices[:, 0], (0, pad)).reshape(_ROWS, 128)
  z2 = jnp.pad(indices[:, 1], (0, pad)).reshape(_ROWS, 128)
  y2 = jnp.pad(indices[:, 2], (0, pad)).reshape(_ROWS, 128)
  x2 = jnp.pad(indices[:, 3], (0, pad)).reshape(_ROWS, 128)

  w0t = jnp.pad(jnp.broadcast_to(W[:, 0, :], (125, 16)), ((0, 3), (0, 0)))
  w1t = jnp.pad(jnp.broadcast_to(W[:, 1, :], (125, 16)), ((0, 3), (0, 0)))

  fa2, fm2, ps2, pg2 = _prep(feats_pad, b2, z2, y2, x2)
  ga, gm = _scatter(ps2.reshape(-1), fa2.reshape(-1), fm2.reshape(-1))
  acc1 = _gather(ga, gm, pg2.reshape(-1), w0t, w1t)
  out = _gate(feats_pad, acc1.reshape(_ROWS, 128))
  return out[:n]
